# baseline (speedup 1.0000x reference)
import jax
import jax.numpy as jnp
from jax import lax
from jax.experimental import pallas as pl
from jax.experimental.pallas import tpu as pltpu

_N, _D = 1_000_000, 64
_C = 2000
_NB = 5
_NCH = _N // _C
_ROUNDS = _NCH // _NB


def _body(x_hbm, o_hbm, in_buf, out_buf, in_sem, out_sem):
    def in_copy(i, b):
        return pltpu.make_async_copy(
            x_hbm.at[pl.ds(i * _C, _C), :], in_buf.at[b], in_sem.at[b])

    def out_copy(i, b):
        return pltpu.make_async_copy(
            out_buf.at[b], o_hbm.at[pl.ds(i * _C, _C), :], out_sem.at[b])

    for b in range(_NB):
        in_copy(b, b).start()

    def stage(i, b, first, last):
        in_copy(i, b).wait()
        if not first:
            out_copy(i - _NB, b).wait()
        out_buf[b] = in_buf[b] + 1.0
        if first and b == 0:
            out_buf[0, 1, :] = jnp.full((_D,), 2.0, dtype=jnp.float32)
        out_copy(i, b).start()
        if not last:
            in_copy(i + _NB, b).start()

    for b in range(_NB):
        stage(b, b, True, False)

    def round_body(r, _):
        for b in range(_NB):
            stage(r * _NB + b, b, False, False)
        return ()

    lax.fori_loop(1, _ROUNDS - 1, round_body, ())

    for b in range(_NB):
        stage((_ROUNDS - 1) * _NB + b, b, False, True)
    for b in range(_NB):
        out_copy((_ROUNDS - 1) * _NB + b, b).wait()


def kernel(x):
    return pl.pallas_call(
        _body,
        in_specs=[pl.BlockSpec(memory_space=pl.ANY)],
        out_specs=pl.BlockSpec(memory_space=pl.ANY),
        out_shape=jax.ShapeDtypeStruct((_N, _D), jnp.float32),
        scratch_shapes=[
            pltpu.VMEM((_NB, _C, _D), jnp.float32),
            pltpu.VMEM((_NB, _C, _D), jnp.float32),
            pltpu.SemaphoreType.DMA((_NB,)),
            pltpu.SemaphoreType.DMA((_NB,)),
        ],
        compiler_params=pltpu.CompilerParams(
            vmem_limit_bytes=100 * 1024 * 1024,
        ),
    )(x)


# speedup vs baseline: 6.3562x; 6.3562x over previous
import jax
import jax.numpy as jnp
from jax.experimental import pallas as pl
from jax.experimental.pallas import tpu as pltpu

_N, _D = 1_000_000, 64
_BC = 32_768


def _body(x_ref, o_ref):
    o_ref[...] = x_ref[...] + 1.0

    @pl.when(pl.program_id(0) == 0)
    def _fix_col1():
        o_ref[:, 1] = jnp.full((_D,), 2.0, dtype=o_ref.dtype)


def kernel(x):
    xt = x.T
    grid = (_N + _BC - 1) // _BC
    out_t = pl.pallas_call(
        _body,
        grid=(grid,),
        in_specs=[pl.BlockSpec((_D, _BC), lambda j: (0, j))],
        out_specs=pl.BlockSpec((_D, _BC), lambda j: (0, j)),
        out_shape=jax.ShapeDtypeStruct((_D, _N), jnp.float32),
    )(xt)
    return out_t.T
